# baseline (speedup 1.0000x reference)
import functools
import math

import jax
import jax.numpy as jnp
from jax import lax
from jax.experimental import pallas as pl
from jax.experimental.pallas import tpu as pltpu
from jax.experimental.pallas import tpu_sc as plsc

N = 10000
E = 320000
D = 128
ALPHA = 0.5
BETAS = (math.log(2.0), math.log(1.5))

NC = 2
NS = 16
NW = NC * NS
CHUNK = 112
NBUF = 3
K = -(-(-(-E // (NW * CHUNK))) // (2 * NBUF)) * (2 * NBUF)
K0 = 180
K1 = 2 * K - K0
E_PAD = NS * (K0 + K1) * CHUNK
NROWS = -(-(N + 1) // 128) * 128

_mesh = plsc.VectorSubcoreMesh(core_axis_name="c", subcore_axis_name="s")


@functools.partial(
    pl.kernel,
    mesh=_mesh,
    out_type=jax.ShapeDtypeStruct((NC, NROWS, D), jnp.float32),
    scratch_types=(
        [pltpu.VMEM((CHUNK,), jnp.int32) for _ in range(2 * NBUF)]
        + [pltpu.VMEM((CHUNK,), jnp.int32) for _ in range(2 * NBUF)]
        + [pltpu.VMEM((NBUF, CHUNK, D), jnp.float32),
           pltpu.VMEM_SHARED((NROWS, D), jnp.float32)]
        + [pltpu.SemaphoreType.DMA for _ in range(3 * NBUF)]
    ),
)
def _sc_agg(h_hbm, src_hbm, dst_hbm, zeros_hbm, out_hbm, *scr):
    isrc = scr[:2 * NBUF]
    idst = scr[2 * NBUF:4 * NBUF]
    rows_v = scr[4 * NBUF]
    acc_sh = scr[4 * NBUF + 1]
    sem_g = scr[4 * NBUF + 2:5 * NBUF + 2]
    sem_i = scr[5 * NBUF + 2:7 * NBUF + 2]
    cid = lax.axis_index("c")
    sid = lax.axis_index("s")
    zrows = NROWS // NS
    pltpu.sync_copy(zeros_hbm.at[pl.ds(sid * zrows, zrows)],
                    acc_sh.at[pl.ds(sid * zrows, zrows)])
    plsc.subcore_barrier()

    def run_pipeline(kc, base):
        def fetch_idx(chunk, j):
            pltpu.make_async_copy(
                src_hbm.at[pl.ds(base + chunk * CHUNK, CHUNK)],
                isrc[j], sem_i[j]).start()
            pltpu.make_async_copy(
                dst_hbm.at[pl.ds(base + chunk * CHUNK, CHUNK)],
                idst[j], sem_i[j]).start()

        def wait_idx(chunk, j):
            pltpu.make_async_copy(
                src_hbm.at[pl.ds(base + chunk * CHUNK, CHUNK)],
                isrc[j], sem_i[j]).wait()
            pltpu.make_async_copy(
                dst_hbm.at[pl.ds(base + chunk * CHUNK, CHUNK)],
                idst[j], sem_i[j]).wait()

        def gather(b, j):
            pltpu.make_async_copy(h_hbm.at[isrc[j]], rows_v.at[b],
                                  sem_g[b]).start()

        def wait_gather(b, j):
            pltpu.make_async_copy(h_hbm.at[isrc[j]], rows_v.at[b],
                                  sem_g[b]).wait()

        def scatter_add(b, j):
            pltpu.sync_copy(rows_v.at[b], acc_sh.at[idst[j]], add=True)

        for b in range(NBUF):
            fetch_idx(b, b)
        for b in range(NBUF):
            fetch_idx(NBUF + b, NBUF + b)
        for b in range(NBUF):
            wait_idx(b, b)
            gather(b, b)

        def half(i, p, last):
            for b in range(NBUF):
                c = i + b
                j = p * NBUF + b
                j2 = (1 - p) * NBUF + b
                wait_gather(b, j)
                scatter_add(b, j)
                if not last:
                    fetch_idx(c + 2 * NBUF, j)
                wait_idx(c + NBUF, j2)
                gather(b, j2)

        @pl.loop(0, kc - 4 * NBUF + 1, step=2 * NBUF)
        def _(i):
            half(i, 0, False)
            half(i + NBUF, 1, False)

        half(kc - 2 * NBUF, 0, True)
        for b in range(NBUF):
            wait_gather(b, NBUF + b)
            scatter_add(b, NBUF + b)

    @pl.when(cid == 0)
    def _():
        run_pipeline(K0, sid * K0 * CHUNK)

    if K1:
        @pl.when(cid == 1)
        def _():
            run_pipeline(K1, (NS * K0 + sid * K1) * CHUNK)

    plsc.subcore_barrier()
    pltpu.sync_copy(acc_sh.at[pl.ds(sid * zrows, zrows)],
                    out_hbm.at[cid, pl.ds(sid * zrows, zrows)])


_ROWBLK = 1000
_GRID = N // _ROWBLK


def _rowspec():
    return pl.BlockSpec((_ROWBLK, D), lambda i: (i, 0))


def _wspec():
    return pl.BlockSpec((D, D), lambda i: (0, 0))


def _mm_body(x_ref, w_ref, o_ref):
    o_ref[...] = jnp.dot(x_ref[...], w_ref[...],
                         preferred_element_type=jnp.float32)


def _tc_matmul(x, w):
    return pl.pallas_call(
        _mm_body,
        grid=(_GRID,),
        in_specs=[_rowspec(), _wspec()],
        out_specs=_rowspec(),
        out_shape=jax.ShapeDtypeStruct((N, D), jnp.float32),
    )(x, w)


def _in_body(p0_ref, p1_ref, b_ref, o_ref):
    o_ref[...] = jnp.maximum(p0_ref[...] + p1_ref[...] + b_ref[...], 0.0)


def _tc_combine_in(p0, p1, b):
    return pl.pallas_call(
        _in_body,
        grid=(_GRID,),
        in_specs=[_rowspec(), _rowspec(),
                  pl.BlockSpec((1, D), lambda i: (0, 0))],
        out_specs=_rowspec(),
        out_shape=jax.ShapeDtypeStruct((N, D), jnp.float32),
    )(p0, p1, b.reshape(1, D))


def _layer_body(p0_ref, p1_ref, x0_ref, w_ref, o_ref, *, beta):
    agg = p0_ref[...] + p1_ref[...]
    out = agg * (1.0 - ALPHA) + ALPHA * x0_ref[...]
    h = (1.0 - beta) * out + beta * jnp.dot(
        out, w_ref[...], preferred_element_type=jnp.float32)
    o_ref[...] = jnp.maximum(h, 0.0)


def _tc_layer(p0, p1, x0, w, beta):
    return pl.pallas_call(
        functools.partial(_layer_body, beta=beta),
        grid=(_GRID,),
        in_specs=[_rowspec(), _rowspec(), _rowspec(), _wspec()],
        out_specs=_rowspec(),
        out_shape=jax.ShapeDtypeStruct((N, D), jnp.float32),
    )(p0, p1, x0, w)


def _layer_out_body(p0_ref, p1_ref, x0_ref, w_ref, wo_ref, o_ref, *, beta):
    agg = p0_ref[...] + p1_ref[...]
    out = agg * (1.0 - ALPHA) + ALPHA * x0_ref[...]
    h = (1.0 - beta) * out + beta * jnp.dot(
        out, w_ref[...], preferred_element_type=jnp.float32)
    h = jnp.maximum(h, 0.0)
    o_ref[...] = jnp.dot(h, wo_ref[...], preferred_element_type=jnp.float32)


def _tc_layer_out(p0, p1, x0, w, beta, w_out):
    return pl.pallas_call(
        functools.partial(_layer_out_body, beta=beta),
        grid=(_GRID,),
        in_specs=[_rowspec(), _rowspec(), _rowspec(), _wspec(), _wspec()],
        out_specs=_rowspec(),
        out_shape=jax.ShapeDtypeStruct((N, D), jnp.float32),
    )(p0, p1, x0, w, w_out)


def _fin_body(p0_ref, p1_ref, b_ref, o_ref):
    o_ref[...] = p0_ref[...] + p1_ref[...] + b_ref[...]


def _tc_final(p0, p1, b):
    return pl.pallas_call(
        _fin_body,
        grid=(_GRID,),
        in_specs=[_rowspec(), _rowspec(),
                  pl.BlockSpec((1, D), lambda i: (0, 0))],
        out_specs=_rowspec(),
        out_shape=jax.ShapeDtypeStruct((N, D), jnp.float32),
    )(p0, p1, b.reshape(1, D))


def _agg(h, src_p, dst_p, zeros_hbm):
    p = _sc_agg(h, src_p, dst_p, zeros_hbm)
    return p[0, :N], p[1, :N]


def kernel(x, edge_index, W_in, b_in, W_layers, W_out, b_out):
    pad = E_PAD - E
    src_p = jnp.concatenate(
        [edge_index[0], jnp.zeros((pad,), jnp.int32)])
    dst_p = jnp.concatenate(
        [edge_index[1],
         N + (jnp.arange(pad, dtype=jnp.int32) % (NROWS - N))])
    zeros_hbm = jnp.zeros((NROWS, D), jnp.float32)

    h = _tc_matmul(x, W_in)
    p0, p1 = _agg(h, src_p, dst_p, zeros_hbm)
    x0 = _tc_combine_in(p0, p1, b_in)
    p0, p1 = _agg(x0, src_p, dst_p, zeros_hbm)
    h = _tc_layer(p0, p1, x0, W_layers[0], BETAS[0])
    p0, p1 = _agg(h, src_p, dst_p, zeros_hbm)
    h = _tc_layer_out(p0, p1, x0, W_layers[1], BETAS[1], W_out)
    p0, p1 = _agg(h, src_p, dst_p, zeros_hbm)
    return _tc_final(p0, p1, b_out)


# speedup vs baseline: 1.2761x; 1.2761x over previous
import functools
import math

import jax
import jax.numpy as jnp
from jax import lax
from jax.experimental import pallas as pl
from jax.experimental.pallas import tpu as pltpu
from jax.experimental.pallas import tpu_sc as plsc

N = 10000
E = 320000
D = 128
ALPHA = 0.5
BETAS = (math.log(2.0), math.log(1.5))

NC = 2
NS = 16
NW = NC * NS
CHUNK = 112
NBUF = 3
K = -(-(-(-E // (NW * CHUNK))) // (2 * NBUF)) * (2 * NBUF)
K0 = 114
K1 = 2 * K - K0
E_PAD = NS * (K0 + K1) * CHUNK
NROWS = -(-(N + 1) // 128) * 128

_mesh = plsc.VectorSubcoreMesh(core_axis_name="c", subcore_axis_name="s")


@functools.partial(
    pl.kernel,
    mesh=_mesh,
    out_type=jax.ShapeDtypeStruct((NC, NROWS, D), jnp.float32),
    scratch_types=(
        [pltpu.VMEM((CHUNK,), jnp.int32) for _ in range(2 * NBUF)]
        + [pltpu.VMEM((CHUNK,), jnp.int32) for _ in range(2 * NBUF)]
        + [pltpu.VMEM((NBUF, CHUNK, D), jnp.float32),
           pltpu.VMEM_SHARED((NROWS, D), jnp.float32)]
        + [pltpu.SemaphoreType.DMA for _ in range(3 * NBUF)]
    ),
)
def _sc_agg(h_hbm, src_hbm, dst_hbm, zeros_hbm, out_hbm, *scr):
    isrc = scr[:2 * NBUF]
    idst = scr[2 * NBUF:4 * NBUF]
    rows_v = scr[4 * NBUF]
    acc_sh = scr[4 * NBUF + 1]
    sem_g = scr[4 * NBUF + 2:5 * NBUF + 2]
    sem_i = scr[5 * NBUF + 2:7 * NBUF + 2]
    cid = lax.axis_index("c")
    sid = lax.axis_index("s")
    zrows = NROWS // NS
    pltpu.sync_copy(zeros_hbm.at[pl.ds(sid * zrows, zrows)],
                    acc_sh.at[pl.ds(sid * zrows, zrows)])
    plsc.subcore_barrier()

    def run_pipeline(kc, base):
        def fetch_idx(chunk, j):
            pltpu.make_async_copy(
                src_hbm.at[pl.ds(base + chunk * CHUNK, CHUNK)],
                isrc[j], sem_i[j]).start()
            pltpu.make_async_copy(
                dst_hbm.at[pl.ds(base + chunk * CHUNK, CHUNK)],
                idst[j], sem_i[j]).start()

        def wait_idx(chunk, j):
            pltpu.make_async_copy(
                src_hbm.at[pl.ds(base + chunk * CHUNK, CHUNK)],
                isrc[j], sem_i[j]).wait()
            pltpu.make_async_copy(
                dst_hbm.at[pl.ds(base + chunk * CHUNK, CHUNK)],
                idst[j], sem_i[j]).wait()

        def gather(b, j):
            pltpu.make_async_copy(h_hbm.at[isrc[j]], rows_v.at[b],
                                  sem_g[b]).start()

        def wait_gather(b, j):
            pltpu.make_async_copy(h_hbm.at[isrc[j]], rows_v.at[b],
                                  sem_g[b]).wait()

        def scatter_add(b, j):
            pltpu.sync_copy(rows_v.at[b], acc_sh.at[idst[j]], add=True)

        for b in range(NBUF):
            fetch_idx(b, b)
        for b in range(NBUF):
            fetch_idx(NBUF + b, NBUF + b)
        for b in range(NBUF):
            wait_idx(b, b)
            gather(b, b)

        def half(i, p, last):
            for b in range(NBUF):
                c = i + b
                j = p * NBUF + b
                j2 = (1 - p) * NBUF + b
                wait_gather(b, j)
                scatter_add(b, j)
                if not last:
                    fetch_idx(c + 2 * NBUF, j)
                wait_idx(c + NBUF, j2)
                gather(b, j2)

        @pl.loop(0, kc - 4 * NBUF + 1, step=2 * NBUF)
        def _(i):
            half(i, 0, False)
            half(i + NBUF, 1, False)

        half(kc - 2 * NBUF, 0, True)
        for b in range(NBUF):
            wait_gather(b, NBUF + b)
            scatter_add(b, NBUF + b)

    @pl.when(cid == 0)
    def _():
        run_pipeline(K0, sid * K0 * CHUNK)

    if K1:
        @pl.when(cid == 1)
        def _():
            run_pipeline(K1, (NS * K0 + sid * K1) * CHUNK)

    plsc.subcore_barrier()
    pltpu.sync_copy(acc_sh.at[pl.ds(sid * zrows, zrows)],
                    out_hbm.at[cid, pl.ds(sid * zrows, zrows)])


_ROWBLK = 1000
_GRID = N // _ROWBLK


def _rowspec():
    return pl.BlockSpec((_ROWBLK, D), lambda i: (i, 0))


def _wspec():
    return pl.BlockSpec((D, D), lambda i: (0, 0))


def _mm_body(x_ref, w_ref, o_ref):
    o_ref[...] = jnp.dot(x_ref[...], w_ref[...],
                         preferred_element_type=jnp.float32)


def _tc_matmul(x, w):
    return pl.pallas_call(
        _mm_body,
        grid=(_GRID,),
        in_specs=[_rowspec(), _wspec()],
        out_specs=_rowspec(),
        out_shape=jax.ShapeDtypeStruct((N, D), jnp.float32),
    )(x, w)


def _in_body(p0_ref, p1_ref, b_ref, o_ref):
    o_ref[...] = jnp.maximum(p0_ref[...] + p1_ref[...] + b_ref[...], 0.0)


def _tc_combine_in(p0, p1, b):
    return pl.pallas_call(
        _in_body,
        grid=(_GRID,),
        in_specs=[_rowspec(), _rowspec(),
                  pl.BlockSpec((1, D), lambda i: (0, 0))],
        out_specs=_rowspec(),
        out_shape=jax.ShapeDtypeStruct((N, D), jnp.float32),
    )(p0, p1, b.reshape(1, D))


def _layer_body(p0_ref, p1_ref, x0_ref, w_ref, o_ref, *, beta):
    agg = p0_ref[...] + p1_ref[...]
    out = agg * (1.0 - ALPHA) + ALPHA * x0_ref[...]
    h = (1.0 - beta) * out + beta * jnp.dot(
        out, w_ref[...], preferred_element_type=jnp.float32)
    o_ref[...] = jnp.maximum(h, 0.0)


def _tc_layer(p0, p1, x0, w, beta):
    return pl.pallas_call(
        functools.partial(_layer_body, beta=beta),
        grid=(_GRID,),
        in_specs=[_rowspec(), _rowspec(), _rowspec(), _wspec()],
        out_specs=_rowspec(),
        out_shape=jax.ShapeDtypeStruct((N, D), jnp.float32),
    )(p0, p1, x0, w)


def _layer_out_body(p0_ref, p1_ref, x0_ref, w_ref, wo_ref, o_ref, *, beta):
    agg = p0_ref[...] + p1_ref[...]
    out = agg * (1.0 - ALPHA) + ALPHA * x0_ref[...]
    h = (1.0 - beta) * out + beta * jnp.dot(
        out, w_ref[...], preferred_element_type=jnp.float32)
    h = jnp.maximum(h, 0.0)
    o_ref[...] = jnp.dot(h, wo_ref[...], preferred_element_type=jnp.float32)


def _tc_layer_out(p0, p1, x0, w, beta, w_out):
    return pl.pallas_call(
        functools.partial(_layer_out_body, beta=beta),
        grid=(_GRID,),
        in_specs=[_rowspec(), _rowspec(), _rowspec(), _wspec(), _wspec()],
        out_specs=_rowspec(),
        out_shape=jax.ShapeDtypeStruct((N, D), jnp.float32),
    )(p0, p1, x0, w, w_out)


def _fin_body(p0_ref, p1_ref, b_ref, o_ref):
    o_ref[...] = p0_ref[...] + p1_ref[...] + b_ref[...]


def _tc_final(p0, p1, b):
    return pl.pallas_call(
        _fin_body,
        grid=(_GRID,),
        in_specs=[_rowspec(), _rowspec(),
                  pl.BlockSpec((1, D), lambda i: (0, 0))],
        out_specs=_rowspec(),
        out_shape=jax.ShapeDtypeStruct((N, D), jnp.float32),
    )(p0, p1, b.reshape(1, D))


def _agg(h, src_p, dst_p, zeros_hbm):
    p = _sc_agg(h, src_p, dst_p, zeros_hbm)
    return p[0, :N], p[1, :N]


def kernel(x, edge_index, W_in, b_in, W_layers, W_out, b_out):
    pad = E_PAD - E
    src_p = jnp.concatenate(
        [edge_index[0], jnp.zeros((pad,), jnp.int32)])
    dst_p = jnp.concatenate(
        [edge_index[1],
         N + (jnp.arange(pad, dtype=jnp.int32) % (NROWS - N))])
    zeros_hbm = jnp.zeros((NROWS, D), jnp.float32)

    h = _tc_matmul(x, W_in)
    p0, p1 = _agg(h, src_p, dst_p, zeros_hbm)
    x0 = _tc_combine_in(p0, p1, b_in)
    p0, p1 = _agg(x0, src_p, dst_p, zeros_hbm)
    h = _tc_layer(p0, p1, x0, W_layers[0], BETAS[0])
    p0, p1 = _agg(h, src_p, dst_p, zeros_hbm)
    h = _tc_layer_out(p0, p1, x0, W_layers[1], BETAS[1], W_out)
    p0, p1 = _agg(h, src_p, dst_p, zeros_hbm)
    return _tc_final(p0, p1, b_out)


# speedup vs baseline: 1.3286x; 1.0412x over previous
import functools
import math

import jax
import jax.numpy as jnp
from jax import lax
from jax.experimental import pallas as pl
from jax.experimental.pallas import tpu as pltpu
from jax.experimental.pallas import tpu_sc as plsc

N = 10000
E = 320000
D = 128
ALPHA = 0.5
BETAS = (math.log(2.0), math.log(1.5))

NC = 2
NS = 16
NW = NC * NS
CHUNK = 112
NBUF = 3
K = -(-(-(-E // (NW * CHUNK))) // (2 * NBUF)) * (2 * NBUF)
K0 = 132
K1 = 2 * K - K0
E_PAD = NS * (K0 + K1) * CHUNK
NROWS = -(-(N + 1) // 128) * 128

_mesh = plsc.VectorSubcoreMesh(core_axis_name="c", subcore_axis_name="s")


@functools.partial(
    pl.kernel,
    mesh=_mesh,
    out_type=jax.ShapeDtypeStruct((NC, NROWS, D), jnp.float32),
    scratch_types=(
        [pltpu.VMEM((CHUNK,), jnp.int32) for _ in range(2 * NBUF)]
        + [pltpu.VMEM((CHUNK,), jnp.int32) for _ in range(2 * NBUF)]
        + [pltpu.VMEM((NBUF, CHUNK, D), jnp.float32),
           pltpu.VMEM_SHARED((NROWS, D), jnp.float32)]
        + [pltpu.SemaphoreType.DMA for _ in range(3 * NBUF)]
    ),
)
def _sc_agg(h_hbm, src_hbm, dst_hbm, zeros_hbm, out_hbm, *scr):
    isrc = scr[:2 * NBUF]
    idst = scr[2 * NBUF:4 * NBUF]
    rows_v = scr[4 * NBUF]
    acc_sh = scr[4 * NBUF + 1]
    sem_g = scr[4 * NBUF + 2:5 * NBUF + 2]
    sem_i = scr[5 * NBUF + 2:7 * NBUF + 2]
    cid = lax.axis_index("c")
    sid = lax.axis_index("s")
    zrows = NROWS // NS
    pltpu.sync_copy(zeros_hbm.at[pl.ds(sid * zrows, zrows)],
                    acc_sh.at[pl.ds(sid * zrows, zrows)])
    plsc.subcore_barrier()

    def run_pipeline(kc, base):
        def fetch_idx(chunk, j):
            pltpu.make_async_copy(
                src_hbm.at[pl.ds(base + chunk * CHUNK, CHUNK)],
                isrc[j], sem_i[j]).start()
            pltpu.make_async_copy(
                dst_hbm.at[pl.ds(base + chunk * CHUNK, CHUNK)],
                idst[j], sem_i[j]).start()

        def wait_idx(chunk, j):
            pltpu.make_async_copy(
                src_hbm.at[pl.ds(base + chunk * CHUNK, CHUNK)],
                isrc[j], sem_i[j]).wait()
            pltpu.make_async_copy(
                dst_hbm.at[pl.ds(base + chunk * CHUNK, CHUNK)],
                idst[j], sem_i[j]).wait()

        def gather(b, j):
            pltpu.make_async_copy(h_hbm.at[isrc[j]], rows_v.at[b],
                                  sem_g[b]).start()

        def wait_gather(b, j):
            pltpu.make_async_copy(h_hbm.at[isrc[j]], rows_v.at[b],
                                  sem_g[b]).wait()

        def scatter_add(b, j):
            pltpu.sync_copy(rows_v.at[b], acc_sh.at[idst[j]], add=True)

        for b in range(NBUF):
            fetch_idx(b, b)
        for b in range(NBUF):
            fetch_idx(NBUF + b, NBUF + b)
        for b in range(NBUF):
            wait_idx(b, b)
            gather(b, b)

        def half(i, p, last):
            for b in range(NBUF):
                c = i + b
                j = p * NBUF + b
                j2 = (1 - p) * NBUF + b
                wait_gather(b, j)
                scatter_add(b, j)
                if not last:
                    fetch_idx(c + 2 * NBUF, j)
                wait_idx(c + NBUF, j2)
                gather(b, j2)

        @pl.loop(0, kc - 4 * NBUF + 1, step=2 * NBUF)
        def _(i):
            half(i, 0, False)
            half(i + NBUF, 1, False)

        half(kc - 2 * NBUF, 0, True)
        for b in range(NBUF):
            wait_gather(b, NBUF + b)
            scatter_add(b, NBUF + b)

    @pl.when(cid == 0)
    def _():
        run_pipeline(K0, sid * K0 * CHUNK)

    if K1:
        @pl.when(cid == 1)
        def _():
            run_pipeline(K1, (NS * K0 + sid * K1) * CHUNK)

    plsc.subcore_barrier()
    pltpu.sync_copy(acc_sh.at[pl.ds(sid * zrows, zrows)],
                    out_hbm.at[cid, pl.ds(sid * zrows, zrows)])


_ROWBLK = 1000
_GRID = N // _ROWBLK


def _rowspec():
    return pl.BlockSpec((_ROWBLK, D), lambda i: (i, 0))


def _wspec():
    return pl.BlockSpec((D, D), lambda i: (0, 0))


def _mm_body(x_ref, w_ref, o_ref):
    o_ref[...] = jnp.dot(x_ref[...], w_ref[...],
                         preferred_element_type=jnp.float32)


def _tc_matmul(x, w):
    return pl.pallas_call(
        _mm_body,
        grid=(_GRID,),
        in_specs=[_rowspec(), _wspec()],
        out_specs=_rowspec(),
        out_shape=jax.ShapeDtypeStruct((N, D), jnp.float32),
    )(x, w)


def _in_body(p0_ref, p1_ref, b_ref, o_ref):
    o_ref[...] = jnp.maximum(p0_ref[...] + p1_ref[...] + b_ref[...], 0.0)


def _tc_combine_in(p0, p1, b):
    return pl.pallas_call(
        _in_body,
        grid=(_GRID,),
        in_specs=[_rowspec(), _rowspec(),
                  pl.BlockSpec((1, D), lambda i: (0, 0))],
        out_specs=_rowspec(),
        out_shape=jax.ShapeDtypeStruct((N, D), jnp.float32),
    )(p0, p1, b.reshape(1, D))


def _layer_body(p0_ref, p1_ref, x0_ref, w_ref, o_ref, *, beta):
    agg = p0_ref[...] + p1_ref[...]
    out = agg * (1.0 - ALPHA) + ALPHA * x0_ref[...]
    h = (1.0 - beta) * out + beta * jnp.dot(
        out, w_ref[...], preferred_element_type=jnp.float32)
    o_ref[...] = jnp.maximum(h, 0.0)


def _tc_layer(p0, p1, x0, w, beta):
    return pl.pallas_call(
        functools.partial(_layer_body, beta=beta),
        grid=(_GRID,),
        in_specs=[_rowspec(), _rowspec(), _rowspec(), _wspec()],
        out_specs=_rowspec(),
        out_shape=jax.ShapeDtypeStruct((N, D), jnp.float32),
    )(p0, p1, x0, w)


def _layer_out_body(p0_ref, p1_ref, x0_ref, w_ref, wo_ref, o_ref, *, beta):
    agg = p0_ref[...] + p1_ref[...]
    out = agg * (1.0 - ALPHA) + ALPHA * x0_ref[...]
    h = (1.0 - beta) * out + beta * jnp.dot(
        out, w_ref[...], preferred_element_type=jnp.float32)
    h = jnp.maximum(h, 0.0)
    o_ref[...] = jnp.dot(h, wo_ref[...], preferred_element_type=jnp.float32)


def _tc_layer_out(p0, p1, x0, w, beta, w_out):
    return pl.pallas_call(
        functools.partial(_layer_out_body, beta=beta),
        grid=(_GRID,),
        in_specs=[_rowspec(), _rowspec(), _rowspec(), _wspec(), _wspec()],
        out_specs=_rowspec(),
        out_shape=jax.ShapeDtypeStruct((N, D), jnp.float32),
    )(p0, p1, x0, w, w_out)


def _fin_body(p0_ref, p1_ref, b_ref, o_ref):
    o_ref[...] = p0_ref[...] + p1_ref[...] + b_ref[...]


def _tc_final(p0, p1, b):
    return pl.pallas_call(
        _fin_body,
        grid=(_GRID,),
        in_specs=[_rowspec(), _rowspec(),
                  pl.BlockSpec((1, D), lambda i: (0, 0))],
        out_specs=_rowspec(),
        out_shape=jax.ShapeDtypeStruct((N, D), jnp.float32),
    )(p0, p1, b.reshape(1, D))


def _agg(h, src_p, dst_p, zeros_hbm):
    p = _sc_agg(h, src_p, dst_p, zeros_hbm)
    return p[0, :N], p[1, :N]


def kernel(x, edge_index, W_in, b_in, W_layers, W_out, b_out):
    pad = E_PAD - E
    src_p = jnp.concatenate(
        [edge_index[0], jnp.zeros((pad,), jnp.int32)])
    dst_p = jnp.concatenate(
        [edge_index[1],
         N + (jnp.arange(pad, dtype=jnp.int32) % (NROWS - N))])
    zeros_hbm = jnp.zeros((NROWS, D), jnp.float32)

    h = _tc_matmul(x, W_in)
    p0, p1 = _agg(h, src_p, dst_p, zeros_hbm)
    x0 = _tc_combine_in(p0, p1, b_in)
    p0, p1 = _agg(x0, src_p, dst_p, zeros_hbm)
    h = _tc_layer(p0, p1, x0, W_layers[0], BETAS[0])
    p0, p1 = _agg(h, src_p, dst_p, zeros_hbm)
    h = _tc_layer_out(p0, p1, x0, W_layers[1], BETAS[1], W_out)
    p0, p1 = _agg(h, src_p, dst_p, zeros_hbm)
    return _tc_final(p0, p1, b_out)


# speedup vs baseline: 1.3638x; 1.0265x over previous
import functools
import math

import jax
import jax.numpy as jnp
from jax import lax
from jax.experimental import pallas as pl
from jax.experimental.pallas import tpu as pltpu
from jax.experimental.pallas import tpu_sc as plsc

N = 10000
E = 320000
D = 128
ALPHA = 0.5
BETAS = (math.log(2.0), math.log(1.5))

NC = 2
NS = 16
NW = NC * NS
CHUNK = 112
NBUF = 3
K = -(-(-(-E // (NW * CHUNK))) // (2 * NBUF)) * (2 * NBUF)
K0 = 144
K1 = 2 * K - K0
E_PAD = NS * (K0 + K1) * CHUNK
NROWS = -(-(N + 1) // 128) * 128

_mesh = plsc.VectorSubcoreMesh(core_axis_name="c", subcore_axis_name="s")


@functools.partial(
    pl.kernel,
    mesh=_mesh,
    out_type=jax.ShapeDtypeStruct((NC, NROWS, D), jnp.float32),
    scratch_types=(
        [pltpu.VMEM((CHUNK,), jnp.int32) for _ in range(2 * NBUF)]
        + [pltpu.VMEM((CHUNK,), jnp.int32) for _ in range(2 * NBUF)]
        + [pltpu.VMEM((NBUF, CHUNK, D), jnp.float32),
           pltpu.VMEM_SHARED((NROWS, D), jnp.float32)]
        + [pltpu.SemaphoreType.DMA for _ in range(3 * NBUF)]
    ),
)
def _sc_agg(h_hbm, src_hbm, dst_hbm, zeros_hbm, out_hbm, *scr):
    isrc = scr[:2 * NBUF]
    idst = scr[2 * NBUF:4 * NBUF]
    rows_v = scr[4 * NBUF]
    acc_sh = scr[4 * NBUF + 1]
    sem_g = scr[4 * NBUF + 2:5 * NBUF + 2]
    sem_i = scr[5 * NBUF + 2:7 * NBUF + 2]
    cid = lax.axis_index("c")
    sid = lax.axis_index("s")
    zrows = NROWS // NS
    pltpu.sync_copy(zeros_hbm.at[pl.ds(sid * zrows, zrows)],
                    acc_sh.at[pl.ds(sid * zrows, zrows)])
    plsc.subcore_barrier()

    def run_pipeline(kc, base):
        def fetch_idx(chunk, j):
            pltpu.make_async_copy(
                src_hbm.at[pl.ds(base + chunk * CHUNK, CHUNK)],
                isrc[j], sem_i[j]).start()
            pltpu.make_async_copy(
                dst_hbm.at[pl.ds(base + chunk * CHUNK, CHUNK)],
                idst[j], sem_i[j]).start()

        def wait_idx(chunk, j):
            pltpu.make_async_copy(
                src_hbm.at[pl.ds(base + chunk * CHUNK, CHUNK)],
                isrc[j], sem_i[j]).wait()
            pltpu.make_async_copy(
                dst_hbm.at[pl.ds(base + chunk * CHUNK, CHUNK)],
                idst[j], sem_i[j]).wait()

        def gather(b, j):
            pltpu.make_async_copy(h_hbm.at[isrc[j]], rows_v.at[b],
                                  sem_g[b]).start()

        def wait_gather(b, j):
            pltpu.make_async_copy(h_hbm.at[isrc[j]], rows_v.at[b],
                                  sem_g[b]).wait()

        def scatter_add(b, j):
            pltpu.sync_copy(rows_v.at[b], acc_sh.at[idst[j]], add=True)

        for b in range(NBUF):
            fetch_idx(b, b)
        for b in range(NBUF):
            fetch_idx(NBUF + b, NBUF + b)
        for b in range(NBUF):
            wait_idx(b, b)
            gather(b, b)

        def half(i, p, last):
            for b in range(NBUF):
                c = i + b
                j = p * NBUF + b
                j2 = (1 - p) * NBUF + b
                wait_gather(b, j)
                scatter_add(b, j)
                if not last:
                    fetch_idx(c + 2 * NBUF, j)
                wait_idx(c + NBUF, j2)
                gather(b, j2)

        @pl.loop(0, kc - 4 * NBUF + 1, step=2 * NBUF)
        def _(i):
            half(i, 0, False)
            half(i + NBUF, 1, False)

        half(kc - 2 * NBUF, 0, True)
        for b in range(NBUF):
            wait_gather(b, NBUF + b)
            scatter_add(b, NBUF + b)

    @pl.when(cid == 0)
    def _():
        run_pipeline(K0, sid * K0 * CHUNK)

    if K1:
        @pl.when(cid == 1)
        def _():
            run_pipeline(K1, (NS * K0 + sid * K1) * CHUNK)

    plsc.subcore_barrier()
    pltpu.sync_copy(acc_sh.at[pl.ds(sid * zrows, zrows)],
                    out_hbm.at[cid, pl.ds(sid * zrows, zrows)])


_ROWBLK = 1000
_GRID = N // _ROWBLK


def _rowspec():
    return pl.BlockSpec((_ROWBLK, D), lambda i: (i, 0))


def _wspec():
    return pl.BlockSpec((D, D), lambda i: (0, 0))


def _mm_body(x_ref, w_ref, o_ref):
    o_ref[...] = jnp.dot(x_ref[...], w_ref[...],
                         preferred_element_type=jnp.float32)


def _tc_matmul(x, w):
    return pl.pallas_call(
        _mm_body,
        grid=(_GRID,),
        in_specs=[_rowspec(), _wspec()],
        out_specs=_rowspec(),
        out_shape=jax.ShapeDtypeStruct((N, D), jnp.float32),
    )(x, w)


def _in_body(p0_ref, p1_ref, b_ref, o_ref):
    o_ref[...] = jnp.maximum(p0_ref[...] + p1_ref[...] + b_ref[...], 0.0)


def _tc_combine_in(p0, p1, b):
    return pl.pallas_call(
        _in_body,
        grid=(_GRID,),
        in_specs=[_rowspec(), _rowspec(),
                  pl.BlockSpec((1, D), lambda i: (0, 0))],
        out_specs=_rowspec(),
        out_shape=jax.ShapeDtypeStruct((N, D), jnp.float32),
    )(p0, p1, b.reshape(1, D))


def _layer_body(p0_ref, p1_ref, x0_ref, w_ref, o_ref, *, beta):
    agg = p0_ref[...] + p1_ref[...]
    out = agg * (1.0 - ALPHA) + ALPHA * x0_ref[...]
    h = (1.0 - beta) * out + beta * jnp.dot(
        out, w_ref[...], preferred_element_type=jnp.float32)
    o_ref[...] = jnp.maximum(h, 0.0)


def _tc_layer(p0, p1, x0, w, beta):
    return pl.pallas_call(
        functools.partial(_layer_body, beta=beta),
        grid=(_GRID,),
        in_specs=[_rowspec(), _rowspec(), _rowspec(), _wspec()],
        out_specs=_rowspec(),
        out_shape=jax.ShapeDtypeStruct((N, D), jnp.float32),
    )(p0, p1, x0, w)


def _layer_out_body(p0_ref, p1_ref, x0_ref, w_ref, wo_ref, o_ref, *, beta):
    agg = p0_ref[...] + p1_ref[...]
    out = agg * (1.0 - ALPHA) + ALPHA * x0_ref[...]
    h = (1.0 - beta) * out + beta * jnp.dot(
        out, w_ref[...], preferred_element_type=jnp.float32)
    h = jnp.maximum(h, 0.0)
    o_ref[...] = jnp.dot(h, wo_ref[...], preferred_element_type=jnp.float32)


def _tc_layer_out(p0, p1, x0, w, beta, w_out):
    return pl.pallas_call(
        functools.partial(_layer_out_body, beta=beta),
        grid=(_GRID,),
        in_specs=[_rowspec(), _rowspec(), _rowspec(), _wspec(), _wspec()],
        out_specs=_rowspec(),
        out_shape=jax.ShapeDtypeStruct((N, D), jnp.float32),
    )(p0, p1, x0, w, w_out)


def _fin_body(p0_ref, p1_ref, b_ref, o_ref):
    o_ref[...] = p0_ref[...] + p1_ref[...] + b_ref[...]


def _tc_final(p0, p1, b):
    return pl.pallas_call(
        _fin_body,
        grid=(_GRID,),
        in_specs=[_rowspec(), _rowspec(),
                  pl.BlockSpec((1, D), lambda i: (0, 0))],
        out_specs=_rowspec(),
        out_shape=jax.ShapeDtypeStruct((N, D), jnp.float32),
    )(p0, p1, b.reshape(1, D))


def _agg(h, src_p, dst_p, zeros_hbm):
    p = _sc_agg(h, src_p, dst_p, zeros_hbm)
    return p[0, :N], p[1, :N]


def kernel(x, edge_index, W_in, b_in, W_layers, W_out, b_out):
    pad = E_PAD - E
    src_p = jnp.concatenate(
        [edge_index[0], jnp.zeros((pad,), jnp.int32)])
    dst_p = jnp.concatenate(
        [edge_index[1],
         N + (jnp.arange(pad, dtype=jnp.int32) % (NROWS - N))])
    zeros_hbm = jnp.zeros((NROWS, D), jnp.float32)

    h = _tc_matmul(x, W_in)
    p0, p1 = _agg(h, src_p, dst_p, zeros_hbm)
    x0 = _tc_combine_in(p0, p1, b_in)
    p0, p1 = _agg(x0, src_p, dst_p, zeros_hbm)
    h = _tc_layer(p0, p1, x0, W_layers[0], BETAS[0])
    p0, p1 = _agg(h, src_p, dst_p, zeros_hbm)
    h = _tc_layer_out(p0, p1, x0, W_layers[1], BETAS[1], W_out)
    p0, p1 = _agg(h, src_p, dst_p, zeros_hbm)
    return _tc_final(p0, p1, b_out)


# speedup vs baseline: 1.4037x; 1.0293x over previous
import functools
import math

import jax
import jax.numpy as jnp
from jax import lax
from jax.experimental import pallas as pl
from jax.experimental.pallas import tpu as pltpu
from jax.experimental.pallas import tpu_sc as plsc

N = 10000
E = 320000
D = 128
ALPHA = 0.5
BETAS = (math.log(2.0), math.log(1.5))

NC = 2
NS = 16
NW = NC * NS
CHUNK = 112
NBUF = 3
K = -(-(-(-E // (NW * CHUNK))) // (2 * NBUF)) * (2 * NBUF)
K0 = 156
K1 = 2 * K - K0
E_PAD = NS * (K0 + K1) * CHUNK
NROWS = -(-(N + 1) // 128) * 128

_mesh = plsc.VectorSubcoreMesh(core_axis_name="c", subcore_axis_name="s")


@functools.partial(
    pl.kernel,
    mesh=_mesh,
    out_type=jax.ShapeDtypeStruct((NC, NROWS, D), jnp.float32),
    scratch_types=(
        [pltpu.VMEM((CHUNK,), jnp.int32) for _ in range(2 * NBUF)]
        + [pltpu.VMEM((CHUNK,), jnp.int32) for _ in range(2 * NBUF)]
        + [pltpu.VMEM((NBUF, CHUNK, D), jnp.float32),
           pltpu.VMEM_SHARED((NROWS, D), jnp.float32)]
        + [pltpu.SemaphoreType.DMA for _ in range(3 * NBUF)]
    ),
)
def _sc_agg(h_hbm, src_hbm, dst_hbm, zeros_hbm, out_hbm, *scr):
    isrc = scr[:2 * NBUF]
    idst = scr[2 * NBUF:4 * NBUF]
    rows_v = scr[4 * NBUF]
    acc_sh = scr[4 * NBUF + 1]
    sem_g = scr[4 * NBUF + 2:5 * NBUF + 2]
    sem_i = scr[5 * NBUF + 2:7 * NBUF + 2]
    cid = lax.axis_index("c")
    sid = lax.axis_index("s")
    zrows = NROWS // NS
    pltpu.sync_copy(zeros_hbm.at[pl.ds(sid * zrows, zrows)],
                    acc_sh.at[pl.ds(sid * zrows, zrows)])
    plsc.subcore_barrier()

    def run_pipeline(kc, base):
        def fetch_idx(chunk, j):
            pltpu.make_async_copy(
                src_hbm.at[pl.ds(base + chunk * CHUNK, CHUNK)],
                isrc[j], sem_i[j]).start()
            pltpu.make_async_copy(
                dst_hbm.at[pl.ds(base + chunk * CHUNK, CHUNK)],
                idst[j], sem_i[j]).start()

        def wait_idx(chunk, j):
            pltpu.make_async_copy(
                src_hbm.at[pl.ds(base + chunk * CHUNK, CHUNK)],
                isrc[j], sem_i[j]).wait()
            pltpu.make_async_copy(
                dst_hbm.at[pl.ds(base + chunk * CHUNK, CHUNK)],
                idst[j], sem_i[j]).wait()

        def gather(b, j):
            pltpu.make_async_copy(h_hbm.at[isrc[j]], rows_v.at[b],
                                  sem_g[b]).start()

        def wait_gather(b, j):
            pltpu.make_async_copy(h_hbm.at[isrc[j]], rows_v.at[b],
                                  sem_g[b]).wait()

        def scatter_add(b, j):
            pltpu.sync_copy(rows_v.at[b], acc_sh.at[idst[j]], add=True)

        for b in range(NBUF):
            fetch_idx(b, b)
        for b in range(NBUF):
            fetch_idx(NBUF + b, NBUF + b)
        for b in range(NBUF):
            wait_idx(b, b)
            gather(b, b)

        def half(i, p, last):
            for b in range(NBUF):
                c = i + b
                j = p * NBUF + b
                j2 = (1 - p) * NBUF + b
                wait_gather(b, j)
                scatter_add(b, j)
                if not last:
                    fetch_idx(c + 2 * NBUF, j)
                wait_idx(c + NBUF, j2)
                gather(b, j2)

        @pl.loop(0, kc - 4 * NBUF + 1, step=2 * NBUF)
        def _(i):
            half(i, 0, False)
            half(i + NBUF, 1, False)

        half(kc - 2 * NBUF, 0, True)
        for b in range(NBUF):
            wait_gather(b, NBUF + b)
            scatter_add(b, NBUF + b)

    @pl.when(cid == 0)
    def _():
        run_pipeline(K0, sid * K0 * CHUNK)

    if K1:
        @pl.when(cid == 1)
        def _():
            run_pipeline(K1, (NS * K0 + sid * K1) * CHUNK)

    plsc.subcore_barrier()
    pltpu.sync_copy(acc_sh.at[pl.ds(sid * zrows, zrows)],
                    out_hbm.at[cid, pl.ds(sid * zrows, zrows)])


_ROWBLK = 1000
_GRID = N // _ROWBLK


def _rowspec():
    return pl.BlockSpec((_ROWBLK, D), lambda i: (i, 0))


def _wspec():
    return pl.BlockSpec((D, D), lambda i: (0, 0))


def _mm_body(x_ref, w_ref, o_ref):
    o_ref[...] = jnp.dot(x_ref[...], w_ref[...],
                         preferred_element_type=jnp.float32)


def _tc_matmul(x, w):
    return pl.pallas_call(
        _mm_body,
        grid=(_GRID,),
        in_specs=[_rowspec(), _wspec()],
        out_specs=_rowspec(),
        out_shape=jax.ShapeDtypeStruct((N, D), jnp.float32),
    )(x, w)


def _in_body(p0_ref, p1_ref, b_ref, o_ref):
    o_ref[...] = jnp.maximum(p0_ref[...] + p1_ref[...] + b_ref[...], 0.0)


def _tc_combine_in(p0, p1, b):
    return pl.pallas_call(
        _in_body,
        grid=(_GRID,),
        in_specs=[_rowspec(), _rowspec(),
                  pl.BlockSpec((1, D), lambda i: (0, 0))],
        out_specs=_rowspec(),
        out_shape=jax.ShapeDtypeStruct((N, D), jnp.float32),
    )(p0, p1, b.reshape(1, D))


def _layer_body(p0_ref, p1_ref, x0_ref, w_ref, o_ref, *, beta):
    agg = p0_ref[...] + p1_ref[...]
    out = agg * (1.0 - ALPHA) + ALPHA * x0_ref[...]
    h = (1.0 - beta) * out + beta * jnp.dot(
        out, w_ref[...], preferred_element_type=jnp.float32)
    o_ref[...] = jnp.maximum(h, 0.0)


def _tc_layer(p0, p1, x0, w, beta):
    return pl.pallas_call(
        functools.partial(_layer_body, beta=beta),
        grid=(_GRID,),
        in_specs=[_rowspec(), _rowspec(), _rowspec(), _wspec()],
        out_specs=_rowspec(),
        out_shape=jax.ShapeDtypeStruct((N, D), jnp.float32),
    )(p0, p1, x0, w)


def _layer_out_body(p0_ref, p1_ref, x0_ref, w_ref, wo_ref, o_ref, *, beta):
    agg = p0_ref[...] + p1_ref[...]
    out = agg * (1.0 - ALPHA) + ALPHA * x0_ref[...]
    h = (1.0 - beta) * out + beta * jnp.dot(
        out, w_ref[...], preferred_element_type=jnp.float32)
    h = jnp.maximum(h, 0.0)
    o_ref[...] = jnp.dot(h, wo_ref[...], preferred_element_type=jnp.float32)


def _tc_layer_out(p0, p1, x0, w, beta, w_out):
    return pl.pallas_call(
        functools.partial(_layer_out_body, beta=beta),
        grid=(_GRID,),
        in_specs=[_rowspec(), _rowspec(), _rowspec(), _wspec(), _wspec()],
        out_specs=_rowspec(),
        out_shape=jax.ShapeDtypeStruct((N, D), jnp.float32),
    )(p0, p1, x0, w, w_out)


def _fin_body(p0_ref, p1_ref, b_ref, o_ref):
    o_ref[...] = p0_ref[...] + p1_ref[...] + b_ref[...]


def _tc_final(p0, p1, b):
    return pl.pallas_call(
        _fin_body,
        grid=(_GRID,),
        in_specs=[_rowspec(), _rowspec(),
                  pl.BlockSpec((1, D), lambda i: (0, 0))],
        out_specs=_rowspec(),
        out_shape=jax.ShapeDtypeStruct((N, D), jnp.float32),
    )(p0, p1, b.reshape(1, D))


def _agg(h, src_p, dst_p, zeros_hbm):
    p = _sc_agg(h, src_p, dst_p, zeros_hbm)
    return p[0, :N], p[1, :N]


def kernel(x, edge_index, W_in, b_in, W_layers, W_out, b_out):
    pad = E_PAD - E
    src_p = jnp.concatenate(
        [edge_index[0], jnp.zeros((pad,), jnp.int32)])
    dst_p = jnp.concatenate(
        [edge_index[1],
         N + (jnp.arange(pad, dtype=jnp.int32) % (NROWS - N))])
    zeros_hbm = jnp.zeros((NROWS, D), jnp.float32)

    h = _tc_matmul(x, W_in)
    p0, p1 = _agg(h, src_p, dst_p, zeros_hbm)
    x0 = _tc_combine_in(p0, p1, b_in)
    p0, p1 = _agg(x0, src_p, dst_p, zeros_hbm)
    h = _tc_layer(p0, p1, x0, W_layers[0], BETAS[0])
    p0, p1 = _agg(h, src_p, dst_p, zeros_hbm)
    h = _tc_layer_out(p0, p1, x0, W_layers[1], BETAS[1], W_out)
    p0, p1 = _agg(h, src_p, dst_p, zeros_hbm)
    return _tc_final(p0, p1, b_out)


# speedup vs baseline: 1.4260x; 1.0158x over previous
import functools
import math

import jax
import jax.numpy as jnp
from jax import lax
from jax.experimental import pallas as pl
from jax.experimental.pallas import tpu as pltpu
from jax.experimental.pallas import tpu_sc as plsc

N = 10000
E = 320000
D = 128
ALPHA = 0.5
BETAS = (math.log(2.0), math.log(1.5))

NC = 2
NS = 16
NW = NC * NS
CHUNK = 112
NBUF = 3
K = -(-(-(-E // (NW * CHUNK))) // (2 * NBUF)) * (2 * NBUF)
K0 = 168
K1 = 2 * K - K0
E_PAD = NS * (K0 + K1) * CHUNK
NROWS = -(-(N + 1) // 128) * 128

_mesh = plsc.VectorSubcoreMesh(core_axis_name="c", subcore_axis_name="s")


@functools.partial(
    pl.kernel,
    mesh=_mesh,
    out_type=jax.ShapeDtypeStruct((NC, NROWS, D), jnp.float32),
    scratch_types=(
        [pltpu.VMEM((CHUNK,), jnp.int32) for _ in range(2 * NBUF)]
        + [pltpu.VMEM((CHUNK,), jnp.int32) for _ in range(2 * NBUF)]
        + [pltpu.VMEM((NBUF, CHUNK, D), jnp.float32),
           pltpu.VMEM_SHARED((NROWS, D), jnp.float32)]
        + [pltpu.SemaphoreType.DMA for _ in range(3 * NBUF)]
    ),
)
def _sc_agg(h_hbm, src_hbm, dst_hbm, zeros_hbm, out_hbm, *scr):
    isrc = scr[:2 * NBUF]
    idst = scr[2 * NBUF:4 * NBUF]
    rows_v = scr[4 * NBUF]
    acc_sh = scr[4 * NBUF + 1]
    sem_g = scr[4 * NBUF + 2:5 * NBUF + 2]
    sem_i = scr[5 * NBUF + 2:7 * NBUF + 2]
    cid = lax.axis_index("c")
    sid = lax.axis_index("s")
    zrows = NROWS // NS
    pltpu.sync_copy(zeros_hbm.at[pl.ds(sid * zrows, zrows)],
                    acc_sh.at[pl.ds(sid * zrows, zrows)])
    plsc.subcore_barrier()

    def run_pipeline(kc, base):
        def fetch_idx(chunk, j):
            pltpu.make_async_copy(
                src_hbm.at[pl.ds(base + chunk * CHUNK, CHUNK)],
                isrc[j], sem_i[j]).start()
            pltpu.make_async_copy(
                dst_hbm.at[pl.ds(base + chunk * CHUNK, CHUNK)],
                idst[j], sem_i[j]).start()

        def wait_idx(chunk, j):
            pltpu.make_async_copy(
                src_hbm.at[pl.ds(base + chunk * CHUNK, CHUNK)],
                isrc[j], sem_i[j]).wait()
            pltpu.make_async_copy(
                dst_hbm.at[pl.ds(base + chunk * CHUNK, CHUNK)],
                idst[j], sem_i[j]).wait()

        def gather(b, j):
            pltpu.make_async_copy(h_hbm.at[isrc[j]], rows_v.at[b],
                                  sem_g[b]).start()

        def wait_gather(b, j):
            pltpu.make_async_copy(h_hbm.at[isrc[j]], rows_v.at[b],
                                  sem_g[b]).wait()

        def scatter_add(b, j):
            pltpu.sync_copy(rows_v.at[b], acc_sh.at[idst[j]], add=True)

        for b in range(NBUF):
            fetch_idx(b, b)
        for b in range(NBUF):
            fetch_idx(NBUF + b, NBUF + b)
        for b in range(NBUF):
            wait_idx(b, b)
            gather(b, b)

        def half(i, p, last):
            for b in range(NBUF):
                c = i + b
                j = p * NBUF + b
                j2 = (1 - p) * NBUF + b
                wait_gather(b, j)
                scatter_add(b, j)
                if not last:
                    fetch_idx(c + 2 * NBUF, j)
                wait_idx(c + NBUF, j2)
                gather(b, j2)

        @pl.loop(0, kc - 4 * NBUF + 1, step=2 * NBUF)
        def _(i):
            half(i, 0, False)
            half(i + NBUF, 1, False)

        half(kc - 2 * NBUF, 0, True)
        for b in range(NBUF):
            wait_gather(b, NBUF + b)
            scatter_add(b, NBUF + b)

    @pl.when(cid == 0)
    def _():
        run_pipeline(K0, sid * K0 * CHUNK)

    if K1:
        @pl.when(cid == 1)
        def _():
            run_pipeline(K1, (NS * K0 + sid * K1) * CHUNK)

    plsc.subcore_barrier()
    pltpu.sync_copy(acc_sh.at[pl.ds(sid * zrows, zrows)],
                    out_hbm.at[cid, pl.ds(sid * zrows, zrows)])


_ROWBLK = 1000
_GRID = N // _ROWBLK


def _rowspec():
    return pl.BlockSpec((_ROWBLK, D), lambda i: (i, 0))


def _wspec():
    return pl.BlockSpec((D, D), lambda i: (0, 0))


def _mm_body(x_ref, w_ref, o_ref):
    o_ref[...] = jnp.dot(x_ref[...], w_ref[...],
                         preferred_element_type=jnp.float32)


def _tc_matmul(x, w):
    return pl.pallas_call(
        _mm_body,
        grid=(_GRID,),
        in_specs=[_rowspec(), _wspec()],
        out_specs=_rowspec(),
        out_shape=jax.ShapeDtypeStruct((N, D), jnp.float32),
    )(x, w)


def _in_body(p0_ref, p1_ref, b_ref, o_ref):
    o_ref[...] = jnp.maximum(p0_ref[...] + p1_ref[...] + b_ref[...], 0.0)


def _tc_combine_in(p0, p1, b):
    return pl.pallas_call(
        _in_body,
        grid=(_GRID,),
        in_specs=[_rowspec(), _rowspec(),
                  pl.BlockSpec((1, D), lambda i: (0, 0))],
        out_specs=_rowspec(),
        out_shape=jax.ShapeDtypeStruct((N, D), jnp.float32),
    )(p0, p1, b.reshape(1, D))


def _layer_body(p0_ref, p1_ref, x0_ref, w_ref, o_ref, *, beta):
    agg = p0_ref[...] + p1_ref[...]
    out = agg * (1.0 - ALPHA) + ALPHA * x0_ref[...]
    h = (1.0 - beta) * out + beta * jnp.dot(
        out, w_ref[...], preferred_element_type=jnp.float32)
    o_ref[...] = jnp.maximum(h, 0.0)


def _tc_layer(p0, p1, x0, w, beta):
    return pl.pallas_call(
        functools.partial(_layer_body, beta=beta),
        grid=(_GRID,),
        in_specs=[_rowspec(), _rowspec(), _rowspec(), _wspec()],
        out_specs=_rowspec(),
        out_shape=jax.ShapeDtypeStruct((N, D), jnp.float32),
    )(p0, p1, x0, w)


def _layer_out_body(p0_ref, p1_ref, x0_ref, w_ref, wo_ref, o_ref, *, beta):
    agg = p0_ref[...] + p1_ref[...]
    out = agg * (1.0 - ALPHA) + ALPHA * x0_ref[...]
    h = (1.0 - beta) * out + beta * jnp.dot(
        out, w_ref[...], preferred_element_type=jnp.float32)
    h = jnp.maximum(h, 0.0)
    o_ref[...] = jnp.dot(h, wo_ref[...], preferred_element_type=jnp.float32)


def _tc_layer_out(p0, p1, x0, w, beta, w_out):
    return pl.pallas_call(
        functools.partial(_layer_out_body, beta=beta),
        grid=(_GRID,),
        in_specs=[_rowspec(), _rowspec(), _rowspec(), _wspec(), _wspec()],
        out_specs=_rowspec(),
        out_shape=jax.ShapeDtypeStruct((N, D), jnp.float32),
    )(p0, p1, x0, w, w_out)


def _fin_body(p0_ref, p1_ref, b_ref, o_ref):
    o_ref[...] = p0_ref[...] + p1_ref[...] + b_ref[...]


def _tc_final(p0, p1, b):
    return pl.pallas_call(
        _fin_body,
        grid=(_GRID,),
        in_specs=[_rowspec(), _rowspec(),
                  pl.BlockSpec((1, D), lambda i: (0, 0))],
        out_specs=_rowspec(),
        out_shape=jax.ShapeDtypeStruct((N, D), jnp.float32),
    )(p0, p1, b.reshape(1, D))


def _agg(h, src_p, dst_p, zeros_hbm):
    p = _sc_agg(h, src_p, dst_p, zeros_hbm)
    return p[0, :N], p[1, :N]


def kernel(x, edge_index, W_in, b_in, W_layers, W_out, b_out):
    pad = E_PAD - E
    src_p = jnp.concatenate(
        [edge_index[0], jnp.zeros((pad,), jnp.int32)])
    dst_p = jnp.concatenate(
        [edge_index[1],
         N + (jnp.arange(pad, dtype=jnp.int32) % (NROWS - N))])
    zeros_hbm = jnp.zeros((NROWS, D), jnp.float32)

    h = _tc_matmul(x, W_in)
    p0, p1 = _agg(h, src_p, dst_p, zeros_hbm)
    x0 = _tc_combine_in(p0, p1, b_in)
    p0, p1 = _agg(x0, src_p, dst_p, zeros_hbm)
    h = _tc_layer(p0, p1, x0, W_layers[0], BETAS[0])
    p0, p1 = _agg(h, src_p, dst_p, zeros_hbm)
    h = _tc_layer_out(p0, p1, x0, W_layers[1], BETAS[1], W_out)
    p0, p1 = _agg(h, src_p, dst_p, zeros_hbm)
    return _tc_final(p0, p1, b_out)
